# baseline (speedup 1.0000x reference)
import functools
import math

import jax
import jax.numpy as jnp
from jax import lax
from jax.experimental import pallas as pl
from jax.experimental.pallas import tpu as pltpu
from jax.experimental.pallas import tpu_sc as plsc

N = 10000
E = 320000
IN = 128
HID = 512
SQD = 4.0

NUM_TILES = 16
CHUNK = 128
SS_CHUNKS = 8
SUPERSTEPS = 20
CHUNKS_PER_TILE = SS_CHUNKS * SUPERSTEPS
E_TILE = CHUNK * CHUNKS_PER_TILE
E_PAD = E_TILE * NUM_TILES
IDX_ROWS = E_PAD // CHUNK
ROWS_PER_TILE = CHUNKS_PER_TILE
ACC_ROWS = 10112
ZROWS = ACC_ROWS // NUM_TILES
WB_ROWS = ZROWS

_ROW_BLOCK = 2000


def _pre_body(feat_ref, wq_ref, wk4_ref, out_ref):
    x = feat_ref[...]
    q = jnp.dot(x, wq_ref[...], preferred_element_type=jnp.float32)
    k4 = jnp.dot(x, wk4_ref[...], preferred_element_type=jnp.float32)
    s = jnp.exp(k4)
    out_ref[0, ...] = q * s
    out_ref[1, ...] = s


def _pre(feat, wqT, wk4T):
    B = _ROW_BLOCK
    return pl.pallas_call(
        _pre_body,
        grid=(N // B,),
        in_specs=[
            pl.BlockSpec((B, IN), lambda i: (i, 0)),
            pl.BlockSpec((IN, IN), lambda i: (0, 0)),
            pl.BlockSpec((IN, IN), lambda i: (0, 0)),
        ],
        out_specs=pl.BlockSpec((2, B, IN), lambda i: (0, i, 0)),
        out_shape=jax.ShapeDtypeStruct((2, N, IN), jnp.float32),
    )(feat, wqT, wk4T)


@functools.partial(
    pl.kernel,
    out_type=jax.ShapeDtypeStruct((2, ACC_ROWS, IN), jnp.float32),
    mesh=plsc.VectorSubcoreMesh(core_axis_name="c", subcore_axis_name="s"),
    scratch_types=[
        pltpu.VMEM((SS_CHUNKS, CHUNK), jnp.int32),
        pltpu.VMEM((SS_CHUNKS, CHUNK), jnp.int32),
        pltpu.VMEM((SS_CHUNKS, CHUNK), jnp.int32),
        pltpu.VMEM((SS_CHUNKS, CHUNK), jnp.int32),
        pltpu.VMEM((CHUNK, IN), jnp.float32),
        pltpu.VMEM((CHUNK, IN), jnp.float32),
        pltpu.VMEM_SHARED((ACC_ROWS, IN), jnp.float32),
        pltpu.SemaphoreType.DMA,
        pltpu.SemaphoreType.DMA,
        pltpu.SemaphoreType.DMA,
        pltpu.SemaphoreType.DMA,
    ],
)
def _edge(t2_ref, src3_ref, dst2_ref, zeros_ref, out_ref,
          idx_sA, idx_dA, idx_sB, idx_dB, rowsA, rowsB, acc,
          gsemA, gsemB, ssemA, ssemB):
    cc = lax.axis_index("c")
    s = lax.axis_index("s")

    pltpu.sync_copy(zeros_ref, acc.at[pl.ds(s * ZROWS, ZROWS)])
    plsc.subcore_barrier()

    rbase = s * ROWS_PER_TILE
    bufs = [(rowsA, gsemA, ssemA), (rowsB, gsemB, ssemB)]

    def wait_gather(idx_row, rX, gX):
        pltpu.make_async_copy(t2_ref.at[idx_row], rX, gX).wait()

    def start_scatter(rY, d_row, sY):
        pltpu.async_copy(rY, acc.at[d_row], sY, add=True)

    def wait_scatter(rX, d_row, sX):
        pltpu.make_async_copy(rX, acc.at[d_row], sX).wait()

    def superstep(t, idx_s, idx_d):
        row0 = rbase + t * SS_CHUNKS
        pltpu.sync_copy(src3_ref.at[cc, pl.ds(row0, SS_CHUNKS)], idx_s)
        pltpu.sync_copy(dst2_ref.at[pl.ds(row0, SS_CHUNKS)], idx_d)
        for m in range(SS_CHUNKS):
            rX, gX, sX = bufs[m % 2]
            pltpu.async_copy(t2_ref.at[idx_s.at[m]], rX, gX)
            wait_gather(idx_s.at[m], rX, gX)
            start_scatter(rX, idx_d.at[m], sX)
            wait_scatter(rX, idx_d.at[m], sX)

    def pair(j, carry):
        superstep(2 * j, idx_sA, idx_dA)
        superstep(2 * j + 1, idx_sB, idx_dB)
        return carry

    lax.fori_loop(0, SUPERSTEPS // 2, pair, 0)
    plsc.subcore_barrier()

    pltpu.sync_copy(acc.at[pl.ds(s * WB_ROWS, WB_ROWS)],
                    out_ref.at[cc, pl.ds(s * WB_ROWS, WB_ROWS)])


def _post_body(acc_ref, feat_ref, g_ref, b_ref, w1_ref, b1_ref, al_ref,
               w2_ref, b2_ref, out_ref):
    num = acc_ref[0, ...]
    den = acc_ref[1, ...]
    g = g_ref[...]
    b = b_ref[...]
    safe = jnp.where(den > 0.0, den, 1.0)
    rst0 = jnp.where(den > 0.0, num / safe, 0.0) + feat_ref[...]
    mu = jnp.mean(rst0, axis=-1, keepdims=True)
    var = jnp.mean((rst0 - mu) ** 2, axis=-1, keepdims=True)
    rst = (rst0 - mu) * lax.rsqrt(var + 1e-5) * g + b
    h = jnp.dot(rst, w1_ref[...], preferred_element_type=jnp.float32) + b1_ref[...]
    h = jnp.where(h >= 0.0, h, al_ref[...] * h)
    z = rst + jnp.dot(h, w2_ref[...], preferred_element_type=jnp.float32) + b2_ref[...]
    mu2 = jnp.mean(z, axis=-1, keepdims=True)
    var2 = jnp.mean((z - mu2) ** 2, axis=-1, keepdims=True)
    out_ref[...] = (z - mu2) * lax.rsqrt(var2 + 1e-5) * g + b


def _post(acc, feat, ln_g, ln_b, w1T, b1, alpha, w2T, b2):
    B = _ROW_BLOCK
    return pl.pallas_call(
        _post_body,
        grid=(N // B,),
        in_specs=[
            pl.BlockSpec((2, B, IN), lambda i: (0, i, 0)),
            pl.BlockSpec((B, IN), lambda i: (i, 0)),
            pl.BlockSpec((1, IN), lambda i: (0, 0)),
            pl.BlockSpec((1, IN), lambda i: (0, 0)),
            pl.BlockSpec((IN, HID), lambda i: (0, 0)),
            pl.BlockSpec((1, HID), lambda i: (0, 0)),
            pl.BlockSpec((1, HID), lambda i: (0, 0)),
            pl.BlockSpec((HID, IN), lambda i: (0, 0)),
            pl.BlockSpec((1, IN), lambda i: (0, 0)),
        ],
        out_specs=pl.BlockSpec((B, IN), lambda i: (i, 0)),
        out_shape=jax.ShapeDtypeStruct((N, IN), jnp.float32),
    )(acc, feat, ln_g.reshape(1, IN), ln_b.reshape(1, IN), w1T,
      b1.reshape(1, HID), alpha.reshape(1, HID), w2T, b2.reshape(1, IN))


def kernel(feat, edge_index, Wq, Wk, Wv, ln_g, ln_b, W1, b1, alpha, W2, b2):
    src = edge_index[0]
    dst = edge_index[1]
    pad = E_PAD - E
    src_p = jnp.concatenate([src, jnp.zeros((pad,), jnp.int32)])
    src3 = jnp.stack([src_p, src_p + N]).reshape(2, IDX_ROWS, CHUNK)
    dst2 = jnp.concatenate([dst, jnp.full((pad,), N, jnp.int32)]).reshape(
        IDX_ROWS, CHUNK)
    zeros = jnp.zeros((ZROWS, IN), jnp.float32)

    t = _pre(feat, Wq.T, (SQD * Wk).T)
    acc = _edge(t.reshape(2 * N, IN), src3, dst2, zeros)
    return _post(acc, feat, ln_g, ln_b, W1.T, b1, alpha, W2.T, b2)


# speedup vs baseline: 1.6745x; 1.6745x over previous
import functools
import math

import jax
import jax.numpy as jnp
from jax import lax
from jax.experimental import pallas as pl
from jax.experimental.pallas import tpu as pltpu
from jax.experimental.pallas import tpu_sc as plsc

N = 10000
E = 320000
IN = 128
HID = 512
SQD = 4.0

NUM_TILES = 16
CHUNK = 128
CHUNKS_PER_TILE = 158
E_TILE = CHUNK * CHUNKS_PER_TILE
E_PAD = E_TILE * NUM_TILES
ACC_ROWS = 10112
ZROWS = ACC_ROWS // NUM_TILES
WB_ROWS = ZROWS

_ROW_BLOCK = 2000


def _pre_body(feat_ref, wq_ref, wk4_ref, out_ref):
    x = feat_ref[...]
    q = jnp.dot(x, wq_ref[...], preferred_element_type=jnp.float32)
    k4 = jnp.dot(x, wk4_ref[...], preferred_element_type=jnp.float32)
    s = jnp.exp(k4)
    out_ref[0, ...] = q * s
    out_ref[1, ...] = s


def _pre(feat, wqT, wk4T):
    B = _ROW_BLOCK
    return pl.pallas_call(
        _pre_body,
        grid=(N // B,),
        in_specs=[
            pl.BlockSpec((B, IN), lambda i: (i, 0)),
            pl.BlockSpec((IN, IN), lambda i: (0, 0)),
            pl.BlockSpec((IN, IN), lambda i: (0, 0)),
        ],
        out_specs=pl.BlockSpec((2, B, IN), lambda i: (0, i, 0)),
        out_shape=jax.ShapeDtypeStruct((2, N, IN), jnp.float32),
    )(feat, wqT, wk4T)


@functools.partial(
    pl.kernel,
    out_type=jax.ShapeDtypeStruct((2, ACC_ROWS, IN), jnp.float32),
    mesh=plsc.VectorSubcoreMesh(core_axis_name="c", subcore_axis_name="s"),
    scratch_types=[
        pltpu.VMEM((CHUNK,), jnp.int32),
        pltpu.VMEM((CHUNK,), jnp.int32),
        pltpu.VMEM((CHUNK,), jnp.int32),
        pltpu.VMEM((CHUNK,), jnp.int32),
        pltpu.VMEM((CHUNK, IN), jnp.float32),
        pltpu.VMEM((CHUNK, IN), jnp.float32),
        pltpu.VMEM_SHARED((ACC_ROWS, IN), jnp.float32),
        pltpu.SemaphoreType.DMA,
        pltpu.SemaphoreType.DMA,
        pltpu.SemaphoreType.DMA,
        pltpu.SemaphoreType.DMA,
    ],
)
def _edge(t2_ref, src2_ref, dst_ref, zeros_ref, out_ref,
          idx_sA, idx_dA, idx_sB, idx_dB, rowsA, rowsB, acc,
          gsemA, gsemB, ssemA, ssemB):
    cc = lax.axis_index("c")
    s = lax.axis_index("s")

    pltpu.sync_copy(zeros_ref, acc.at[pl.ds(s * ZROWS, ZROWS)])
    plsc.subcore_barrier()

    ebase = s * E_TILE

    def load_idx(e0, idx_s, idx_d):
        pltpu.sync_copy(src2_ref.at[cc, pl.ds(e0, CHUNK)], idx_s)
        pltpu.sync_copy(dst_ref.at[pl.ds(e0, CHUNK)], idx_d)

    def start_gather(idx_s, rX, gX):
        pltpu.async_copy(t2_ref.at[idx_s], rX, gX)

    def wait_gather(idx_s, rX, gX):
        pltpu.make_async_copy(t2_ref.at[idx_s], rX, gX).wait()

    def start_scatter(rY, idx_d, sY):
        pltpu.async_copy(rY, acc.at[idx_d], sY, add=True)

    def wait_scatter(rX, idx_d, sX):
        pltpu.make_async_copy(rX, acc.at[idx_d], sX).wait()

    load_idx(ebase, idx_sA, idx_dA)
    start_gather(idx_sA, rowsA, gsemA)
    load_idx(ebase + CHUNK, idx_sB, idx_dB)
    start_gather(idx_sB, rowsB, gsemB)
    wait_gather(idx_sA, rowsA, gsemA)
    start_scatter(rowsA, idx_dA, ssemA)

    def pair(j, carry):
        e0 = ebase + (2 * j + 2) * CHUNK
        wait_scatter(rowsA, idx_dA, ssemA)
        load_idx(e0, idx_sA, idx_dA)
        start_gather(idx_sA, rowsA, gsemA)
        wait_gather(idx_sB, rowsB, gsemB)
        start_scatter(rowsB, idx_dB, ssemB)
        wait_scatter(rowsB, idx_dB, ssemB)
        load_idx(e0 + CHUNK, idx_sB, idx_dB)
        start_gather(idx_sB, rowsB, gsemB)
        wait_gather(idx_sA, rowsA, gsemA)
        start_scatter(rowsA, idx_dA, ssemA)
        return carry

    lax.fori_loop(0, CHUNKS_PER_TILE // 2 - 1, pair, 0)

    wait_gather(idx_sB, rowsB, gsemB)
    start_scatter(rowsB, idx_dB, ssemB)
    wait_scatter(rowsA, idx_dA, ssemA)
    wait_scatter(rowsB, idx_dB, ssemB)
    plsc.subcore_barrier()

    pltpu.sync_copy(acc.at[pl.ds(s * WB_ROWS, WB_ROWS)],
                    out_ref.at[cc, pl.ds(s * WB_ROWS, WB_ROWS)])


def _post_body(acc_ref, feat_ref, g_ref, b_ref, w1_ref, b1_ref, al_ref,
               w2_ref, b2_ref, out_ref):
    num = acc_ref[0, ...]
    den = acc_ref[1, ...]
    g = g_ref[...]
    b = b_ref[...]
    safe = jnp.where(den > 0.0, den, 1.0)
    rst0 = jnp.where(den > 0.0, num / safe, 0.0) + feat_ref[...]
    mu = jnp.mean(rst0, axis=-1, keepdims=True)
    var = jnp.mean((rst0 - mu) ** 2, axis=-1, keepdims=True)
    rst = (rst0 - mu) * lax.rsqrt(var + 1e-5) * g + b
    h = jnp.dot(rst, w1_ref[...], preferred_element_type=jnp.float32) + b1_ref[...]
    h = jnp.where(h >= 0.0, h, al_ref[...] * h)
    z = rst + jnp.dot(h, w2_ref[...], preferred_element_type=jnp.float32) + b2_ref[...]
    mu2 = jnp.mean(z, axis=-1, keepdims=True)
    var2 = jnp.mean((z - mu2) ** 2, axis=-1, keepdims=True)
    out_ref[...] = (z - mu2) * lax.rsqrt(var2 + 1e-5) * g + b


def _post(acc, feat, ln_g, ln_b, w1T, b1, alpha, w2T, b2):
    B = _ROW_BLOCK
    return pl.pallas_call(
        _post_body,
        grid=(N // B,),
        in_specs=[
            pl.BlockSpec((2, B, IN), lambda i: (0, i, 0)),
            pl.BlockSpec((B, IN), lambda i: (i, 0)),
            pl.BlockSpec((1, IN), lambda i: (0, 0)),
            pl.BlockSpec((1, IN), lambda i: (0, 0)),
            pl.BlockSpec((IN, HID), lambda i: (0, 0)),
            pl.BlockSpec((1, HID), lambda i: (0, 0)),
            pl.BlockSpec((1, HID), lambda i: (0, 0)),
            pl.BlockSpec((HID, IN), lambda i: (0, 0)),
            pl.BlockSpec((1, IN), lambda i: (0, 0)),
        ],
        out_specs=pl.BlockSpec((B, IN), lambda i: (i, 0)),
        out_shape=jax.ShapeDtypeStruct((N, IN), jnp.float32),
    )(acc, feat, ln_g.reshape(1, IN), ln_b.reshape(1, IN), w1T,
      b1.reshape(1, HID), alpha.reshape(1, HID), w2T, b2.reshape(1, IN))


def kernel(feat, edge_index, Wq, Wk, Wv, ln_g, ln_b, W1, b1, alpha, W2, b2):
    src = edge_index[0]
    dst = edge_index[1]
    pad = E_PAD - E
    src_p = jnp.concatenate([src, jnp.zeros((pad,), jnp.int32)])
    src2 = jnp.stack([src_p, src_p + N])
    dst_p = jnp.concatenate([dst, jnp.full((pad,), N, jnp.int32)])
    zeros = jnp.zeros((ZROWS, IN), jnp.float32)

    t = _pre(feat, Wq.T, (SQD * Wk).T)
    acc = _edge(t.reshape(2 * N, IN), src2, dst_p, zeros)
    return _post(acc, feat, ln_g, ln_b, W1.T, b1, alpha, W2.T, b2)
